# baseline (speedup 1.0000x reference)
import jax
import jax.numpy as jnp
from jax import lax
from jax.experimental import pallas as pl
from jax.experimental.pallas import tpu as pltpu
from jax.experimental.pallas import tpu_sc as plsc

VOCAB_SIZE = 100000
MODEL_DIM = 128
CODEBOOK_SIZE = 1000000
NGRAM = 4
MULTIPLIERS = (911382323, 972663749, 97266353, 19260817)

_M1000 = tuple((1000 * m) % CODEBOOK_SIZE for m in MULTIPLIERS)
_M1 = tuple(m % CODEBOOK_SIZE for m in MULTIPLIERS)

_B, _T = 4, 8192
_TOKENS = _B * _T
_NW = 32
_PER_W = _TOKENS // _NW
_CHUNK = 128
_NCHUNK = _PER_W // _CHUNK
_CH_PER_ROW = _T // _PER_W
_HALO = 16
_NBUF = 6
_DIST = 3


def _i32(x):
    return jnp.asarray(x, jnp.int32)


def _splat(v):
    return jnp.full((16,), v, jnp.int32)


def _gather_body(ids_hbm, table_hbm, out_hbm,
                 ids_ext, idx_v, mix_v, rows, gsem, osem):
    wid = lax.axis_index("s") * 2 + lax.axis_index("c")
    base = wid * _PER_W
    halo_src = pl.multiple_of(
        jnp.maximum(base - _HALO, 0).astype(jnp.int32), _HALO)
    pltpu.sync_copy(ids_hbm.at[pl.ds(halo_src, _HALO)],
                    ids_ext.at[pl.ds(0, _HALO)])
    base32 = pl.multiple_of(base.astype(jnp.int32), _PER_W)
    pltpu.sync_copy(ids_hbm.at[pl.ds(base32, _PER_W)],
                    ids_ext.at[pl.ds(_HALO, _PER_W)])
    pltpu.sync_copy(ids_hbm.at[pl.ds(_i32(_TOKENS), _HALO)], mix_v)
    m = lax.bitcast_convert_type(mix_v[...], jnp.float32)
    need_scale = m[0] != jnp.asarray(1.0, jnp.float32)
    edge_i = ((wid % _CH_PER_ROW) == 0).astype(jnp.int32)
    edge_v = jnp.broadcast_to(edge_i, (16,))
    lane = lax.broadcasted_iota(jnp.int32, (16,), 0)

    k1000 = _splat(1000)
    kmod = _splat(CODEBOOK_SIZE)
    km1000 = [_splat(v) for v in _M1000[:NGRAM - 1]]
    km1 = [_splat(v) for v in _M1[:NGRAM - 1]]
    koff = [_splat(o) for o in range(1, NGRAM)]
    inv1000 = jnp.full((16,), 1.0 / 1000.0, jnp.float32)
    inv1e6 = jnp.full((16,), 1.0 / float(CODEBOOK_SIZE), jnp.float32)
    half = jnp.full((16,), 0.5, jnp.float32)
    halff = jnp.full((16,), 0.5, jnp.float32)
    k31 = jnp.full((16,), 31, jnp.uint32)

    def _mod1e6(x):
        q = (x.astype(jnp.float32) * inv1e6 - halff).astype(jnp.int32)
        t = x - q * kmod - kmod
        return t + lax.shift_right_logical(t, k31.astype(jnp.int32)) * kmod

    first_tok = lax.gather(
        ids_ext[pl.ds(_HALO, 16)], jnp.zeros((16, 1), jnp.int32),
        dimension_numbers=lax.GatherDimensionNumbers(
            offset_dims=(), collapsed_slice_dims=(0,),
            start_index_map=(0,)),
        slice_sizes=(1,),
        mode=lax.GatherScatterMode.PROMISE_IN_BOUNDS)
    t13 = lane - _splat(13)
    ge13 = _splat(1) - lax.shift_right_logical(t13, k31.astype(jnp.int32))
    sel = edge_v * ge13
    halo = ids_ext[pl.ds(0, 16)]
    ids_ext[pl.ds(0, 16)] = halo + (first_tok - halo) * sel

    def hash_vreg(cdyn, kdyn, p0):
        x = ids_ext[pl.ds(p0, 16)]
        codes = x
        for o in range(1, NGRAM):
            s = ids_ext[pl.ds(p0 - o, 16)]
            a = ((codes.astype(jnp.float32) + half) * inv1000) \
                .astype(jnp.int32)
            b = codes - a * k1000
            codes = _mod1e6(a * km1000[o - 1] + b * km1[o - 1] + s
                            + koff[o - 1])
        idx_v[cdyn, pl.ds(kdyn * 16, 16)] = codes

    def hash_chunk_dyn(cd, lo):
        def body(k, carry):
            hash_vreg(cd, k, _HALO + _CHUNK * cd + 16 * k)
            return carry

        lax.fori_loop(lo, _i32(_CHUNK // 16), body, _i32(0))

    def start_gather(cd, bd):
        return pltpu.async_copy(
            table_hbm.at[idx_v.at[cd]], rows.at[bd], gsem.at[bd])

    def wait_gather(cd, bd):
        pltpu.make_async_copy(
            table_hbm.at[idx_v.at[cd]], rows.at[bd], gsem.at[bd]).wait()

    def start_write(cd, bd):
        off = (base + cd * _CHUNK).astype(jnp.int32)
        return pltpu.async_copy(
            rows.at[bd], out_hbm.at[pl.ds(off, _CHUNK)], osem.at[bd])

    def wait_write(cd, bd):
        off = (base + cd * _CHUNK).astype(jnp.int32)
        pltpu.make_async_copy(
            rows.at[bd], out_hbm.at[pl.ds(off, _CHUNK)], osem.at[bd]).wait()

    hash_chunk_dyn(_i32(0), _i32(0))
    start_gather(_i32(0), _i32(0))
    hash_chunk_dyn(_i32(1), _i32(0))
    start_gather(_i32(1), _i32(1))
    hash_chunk_dyn(_i32(2), _i32(0))
    start_gather(_i32(2), _i32(2))

    def ring(c, carry):
        b = lax.rem(c, _i32(_NBUF))

        @pl.when(c < _i32(_NCHUNK - _DIST))
        def _advance():
            nb = lax.rem(c + _DIST, _i32(_NBUF))

            @pl.when(c >= _i32(_DIST))
            def _drain():
                wait_write(c - _DIST, nb)

            hash_chunk_dyn(c + _DIST, _i32(0))
            start_gather(c + _DIST, nb)

        wait_gather(c, b)

        @pl.when(need_scale)
        def _scale():
            def body(r, carry2):
                def kbody(k, carry3):
                    sl = pl.ds(k * 16, 16)
                    rows[b, r, sl] = rows[b, r, sl] * m
                    return carry3

                lax.fori_loop(_i32(0), _i32(MODEL_DIM // 16), kbody,
                              _i32(0))
                return carry2

            lax.fori_loop(_i32(0), _i32(_CHUNK), body, _i32(0))

        start_write(c, b)
        return carry

    lax.fori_loop(_i32(0), _i32(_NCHUNK), ring, _i32(0))
    for c in range(_NCHUNK - _NBUF, _NCHUNK):
        wait_write(_i32(c), _i32(c % _NBUF))


_gather_call = pl.kernel(
    _gather_body,
    mesh=plsc.VectorSubcoreMesh(core_axis_name="c", subcore_axis_name="s"),
    out_type=jax.ShapeDtypeStruct((_TOKENS, MODEL_DIM), jnp.float32),
    scratch_types=[
        pltpu.VMEM((_HALO + _PER_W,), jnp.int32),
        pltpu.VMEM((_NCHUNK, _CHUNK), jnp.int32),
        pltpu.VMEM((16,), jnp.int32),
        pltpu.VMEM((_NBUF, _CHUNK, MODEL_DIM), jnp.float32),
        pltpu.SemaphoreType.DMA((_NBUF,)),
        pltpu.SemaphoreType.DMA((_NBUF,)),
    ],
)


def kernel(input_ids, emb_weight, mix):
    ids = input_ids.astype(jnp.int32).reshape(_TOKENS)
    mix_bits = jnp.broadcast_to(
        lax.bitcast_convert_type(mix.astype(jnp.float32), jnp.int32), (16,))
    packed = jnp.concatenate([ids, mix_bits])
    out = _gather_call(packed, emb_weight)
    return out.reshape(_B, _T, MODEL_DIM)


# speedup vs baseline: 1.0027x; 1.0027x over previous
import jax
import jax.numpy as jnp
from jax import lax
from jax.experimental import pallas as pl
from jax.experimental.pallas import tpu as pltpu
from jax.experimental.pallas import tpu_sc as plsc

VOCAB_SIZE = 100000
MODEL_DIM = 128
CODEBOOK_SIZE = 1000000
NGRAM = 4
MULTIPLIERS = (911382323, 972663749, 97266353, 19260817)

_M1000 = tuple((1000 * m) % CODEBOOK_SIZE for m in MULTIPLIERS)
_M1 = tuple(m % CODEBOOK_SIZE for m in MULTIPLIERS)

_B, _T = 4, 8192
_TOKENS = _B * _T
_NW = 32
_PER_W = _TOKENS // _NW
_CHUNK = 128
_NCHUNK = _PER_W // _CHUNK
_CH_PER_ROW = _T // _PER_W
_HALO = 16
_NBUF = 6
_DIST = 3


def _i32(x):
    return jnp.asarray(x, jnp.int32)


def _splat(v):
    return jnp.full((16,), v, jnp.int32)


def _gather_body(ids_hbm, table_hbm, out_hbm,
                 ids_ext, idx_v, mix_v, rows, gsem, osem):
    wid = lax.axis_index("s") * 2 + lax.axis_index("c")
    base = wid * _PER_W
    halo_src = pl.multiple_of(
        jnp.maximum(base - _HALO, 0).astype(jnp.int32), _HALO)
    pltpu.sync_copy(ids_hbm.at[pl.ds(halo_src, _HALO)],
                    ids_ext.at[pl.ds(0, _HALO)])
    base32 = pl.multiple_of(base.astype(jnp.int32), _PER_W)
    pltpu.sync_copy(ids_hbm.at[pl.ds(base32, _PER_W)],
                    ids_ext.at[pl.ds(_HALO, _PER_W)])
    pltpu.sync_copy(ids_hbm.at[pl.ds(_i32(_TOKENS), _HALO)], mix_v)
    m = lax.bitcast_convert_type(mix_v[...], jnp.float32)
    need_scale = m[0] != jnp.asarray(1.0, jnp.float32)
    edge_i = ((wid % _CH_PER_ROW) == 0).astype(jnp.int32)
    edge_v = jnp.broadcast_to(edge_i, (16,))
    lane = lax.broadcasted_iota(jnp.int32, (16,), 0)

    k1000 = _splat(1000)
    kmod = _splat(CODEBOOK_SIZE)
    km1000 = [_splat(v) for v in _M1000[:NGRAM - 1]]
    km1 = [_splat(v) for v in _M1[:NGRAM - 1]]
    koff = [_splat(o) for o in range(1, NGRAM)]
    inv1000 = jnp.full((16,), 1.0 / 1000.0, jnp.float32)
    inv1e6 = jnp.full((16,), 1.0 / float(CODEBOOK_SIZE), jnp.float32)
    half = jnp.full((16,), 0.5, jnp.float32)
    k31 = jnp.full((16,), 31, jnp.uint32)

    def _mod1e6(x):
        q = (x.astype(jnp.float32) * inv1e6 - half).astype(jnp.int32)
        t = x - q * kmod - kmod
        return t + lax.shift_right_logical(t, k31.astype(jnp.int32)) * kmod

    first_tok = lax.gather(
        ids_ext[pl.ds(_HALO, 16)], jnp.zeros((16, 1), jnp.int32),
        dimension_numbers=lax.GatherDimensionNumbers(
            offset_dims=(), collapsed_slice_dims=(0,),
            start_index_map=(0,)),
        slice_sizes=(1,),
        mode=lax.GatherScatterMode.PROMISE_IN_BOUNDS)
    t13 = lane - _splat(13)
    ge13 = _splat(1) - lax.shift_right_logical(t13, k31.astype(jnp.int32))
    sel = edge_v * ge13
    halo = ids_ext[pl.ds(0, 16)]
    ids_ext[pl.ds(0, 16)] = halo + (first_tok - halo) * sel

    def hash_vreg(cdyn, kdyn, p0):
        x = ids_ext[pl.ds(p0, 16)]
        codes = x
        for o in range(1, NGRAM):
            s = ids_ext[pl.ds(p0 - o, 16)]
            a = ((codes.astype(jnp.float32) + half) * inv1000) \
                .astype(jnp.int32)
            b = codes - a * k1000
            codes = _mod1e6(a * km1000[o - 1] + b * km1[o - 1] + s
                            + koff[o - 1])
        idx_v[cdyn, pl.ds(kdyn * 16, 16)] = codes

    def hash_chunk_dyn(cd, lo):
        def body(k, carry):
            hash_vreg(cd, k, _HALO + _CHUNK * cd + 16 * k)
            return carry

        lax.fori_loop(lo, _i32(_CHUNK // 16), body, _i32(0))

    def start_gather(cd, bd):
        return pltpu.async_copy(
            table_hbm.at[idx_v.at[cd]], rows.at[bd], gsem.at[bd])

    def wait_gather(cd, bd):
        pltpu.make_async_copy(
            table_hbm.at[idx_v.at[cd]], rows.at[bd], gsem.at[bd]).wait()

    def start_write(cd, bd):
        off = (base + cd * _CHUNK).astype(jnp.int32)
        return pltpu.async_copy(
            rows.at[bd], out_hbm.at[pl.ds(off, _CHUNK)], osem.at[bd])

    def wait_write(cd, bd):
        off = (base + cd * _CHUNK).astype(jnp.int32)
        pltpu.make_async_copy(
            rows.at[bd], out_hbm.at[pl.ds(off, _CHUNK)], osem.at[bd]).wait()

    hash_chunk_dyn(_i32(0), _i32(0))
    start_gather(_i32(0), _i32(0))
    hash_chunk_dyn(_i32(1), _i32(0))
    start_gather(_i32(1), _i32(1))
    hash_chunk_dyn(_i32(2), _i32(0))
    start_gather(_i32(2), _i32(2))

    def ring(c, carry):
        b = lax.rem(c, _i32(_NBUF))

        @pl.when(c < _i32(_NCHUNK - _DIST))
        def _advance():
            nb = lax.rem(c + _DIST, _i32(_NBUF))

            @pl.when(c >= _i32(_DIST))
            def _drain():
                wait_write(c - _DIST, nb)

            hash_chunk_dyn(c + _DIST, _i32(0))
            start_gather(c + _DIST, nb)

        wait_gather(c, b)

        @pl.when(need_scale)
        def _scale():
            def body(r, carry2):
                def kbody(k, carry3):
                    sl = pl.ds(k * 16, 16)
                    rows[b, r, sl] = rows[b, r, sl] * m
                    return carry3

                lax.fori_loop(_i32(0), _i32(MODEL_DIM // 16), kbody,
                              _i32(0))
                return carry2

            lax.fori_loop(_i32(0), _i32(_CHUNK), body, _i32(0))

        start_write(c, b)
        return carry

    lax.fori_loop(_i32(0), _i32(_NCHUNK), ring, _i32(0))
    for c in range(_NCHUNK - _NBUF, _NCHUNK):
        wait_write(_i32(c), _i32(c % _NBUF))


_gather_call = pl.kernel(
    _gather_body,
    mesh=plsc.VectorSubcoreMesh(core_axis_name="c", subcore_axis_name="s"),
    out_type=jax.ShapeDtypeStruct((_TOKENS, MODEL_DIM), jnp.float32),
    scratch_types=[
        pltpu.VMEM((_HALO + _PER_W,), jnp.int32),
        pltpu.VMEM((_NCHUNK, _CHUNK), jnp.int32),
        pltpu.VMEM((16,), jnp.int32),
        pltpu.VMEM((_NBUF, _CHUNK, MODEL_DIM), jnp.float32),
        pltpu.SemaphoreType.DMA((_NBUF,)),
        pltpu.SemaphoreType.DMA((_NBUF,)),
    ],
)


def kernel(input_ids, emb_weight, mix):
    ids = input_ids.astype(jnp.int32).reshape(_TOKENS)
    mix_bits = jnp.broadcast_to(
        lax.bitcast_convert_type(mix.astype(jnp.float32), jnp.int32), (16,))
    packed = jnp.concatenate([ids, mix_bits])
    out = _gather_call(packed, emb_weight)
    return out.reshape(_B, _T, MODEL_DIM)
